# Initial kernel scaffold; baseline (speedup 1.0000x reference)
import jax
import jax.numpy as jnp
from jax.experimental import pallas as pl


def kernel(data_input, neighbor_indices):
    raise NotImplementedError("write your pallas kernel here")


# speedup vs baseline: 1.1591x; 1.1591x over previous
import functools

import jax
import jax.numpy as jnp
from jax import lax
from jax.experimental import pallas as pl
from jax.experimental.pallas import tpu as pltpu
from jax.experimental.pallas import tpu_sc as plsc

N = 8192
K1 = 17
NC, NS, L = 2, 16, 16
NW = NC * NS
RPW = N // NW
E = K1 * RPW
CH = 128
NCHUNK = E // CH


def _sc_row_sums(data_flat, idx_t):
    mesh = plsc.VectorSubcoreMesh(
        core_axis_name="c", subcore_axis_name="s",
        num_cores=NC, num_subcores=NS)

    @functools.partial(
        pl.kernel,
        out_type=jax.ShapeDtypeStruct((N,), jnp.float32),
        mesh=mesh,
        scratch_types=[
            pltpu.VMEM((K1, RPW), jnp.int32),
            pltpu.VMEM((E,), jnp.int32),
            pltpu.VMEM((E,), jnp.float32),
            pltpu.VMEM((RPW,), jnp.float32),
            pltpu.SemaphoreType.DMA,
        ],
    )
    def rowsum_kernel(data_hbm, idx_hbm, out_hbm, cols_v, flat_v, gath_v,
                      sums_v, sem):
        wid = lax.axis_index("s") * NC + lax.axis_index("c")
        base = wid * RPW

        pltpu.sync_copy(idx_hbm.at[:, pl.ds(base, RPW)], cols_v)

        def build_chunk(c, carry):
            roff = (base + c * L + lax.iota(jnp.int32, L)) * N

            def build_j(j, carry):
                colv = cols_v[j, pl.ds(c * L, L)]
                flat_v[pl.ds(j * RPW + c * L, L)] = colv + roff
                return carry

            return lax.fori_loop(0, K1, build_j, carry)

        lax.fori_loop(0, RPW // L, build_chunk, 0)

        descs = [
            pltpu.async_copy(
                data_hbm.at[flat_v.at[pl.ds(m * CH, CH)]],
                gath_v.at[pl.ds(m * CH, CH)], sem)
            for m in range(NCHUNK)
        ]
        for d in descs:
            d.wait()

        def reduce_chunk(c, carry):
            def add_j(j, acc):
                return acc + gath_v[pl.ds(j * RPW + c * L, L)]

            acc = lax.fori_loop(1, K1, add_j, gath_v[pl.ds(c * L, L)])
            sums_v[pl.ds(c * L, L)] = acc
            return carry

        lax.fori_loop(0, RPW // L, reduce_chunk, 0)

        pltpu.sync_copy(sums_v, out_hbm.at[pl.ds(base, RPW)])

    return rowsum_kernel(data_flat, idx_t)


def _tc_softmax(a_raw):
    def body(x_ref, alpha_ref):
        x = x_ref[...]
        m = jnp.max(x)
        e = jnp.exp(x - m)
        alpha_ref[...] = e / jnp.sum(e)

    alpha = pl.pallas_call(
        body,
        out_shape=jax.ShapeDtypeStruct((8, N // 8), jnp.float32),
    )(a_raw.reshape(8, N // 8))
    return alpha.reshape(N)


def kernel(data_input, neighbor_indices):
    idx = neighbor_indices[:, :K1].astype(jnp.int32)
    idx_t = idx.T.reshape(K1, N)
    data_flat = data_input.reshape(N * N)
    a_raw = _sc_row_sums(data_flat, idx_t)
    alpha = _tc_softmax(a_raw)
    return (alpha, a_raw)


# speedup vs baseline: 7.3938x; 6.3787x over previous
import functools

import jax
import jax.numpy as jnp
from jax import lax
from jax.experimental import pallas as pl
from jax.experimental.pallas import tpu as pltpu
from jax.experimental.pallas import tpu_sc as plsc

N = 8192
K1 = 17
NC, NS, L = 2, 16, 16
NW = NC * NS
RPW = N // NW
E = K1 * RPW
CH = 128
NCHUNK = E // CH


def _sc_row_sums(data_phys, idx_t):
    mesh = plsc.VectorSubcoreMesh(
        core_axis_name="c", subcore_axis_name="s",
        num_cores=NC, num_subcores=NS)

    @functools.partial(
        pl.kernel,
        out_type=jax.ShapeDtypeStruct((N,), jnp.float32),
        mesh=mesh,
        scratch_types=[
            pltpu.VMEM((K1, RPW), jnp.int32),
            pltpu.VMEM((E,), jnp.int32),
            pltpu.VMEM((E,), jnp.float32),
            pltpu.VMEM((RPW,), jnp.float32),
            pltpu.SemaphoreType.DMA,
        ],
    )
    def rowsum_kernel(data_hbm, idx_hbm, out_hbm, cols_v, flat_v, gath_v,
                      sums_v, sem):
        wid = lax.axis_index("s") * NC + lax.axis_index("c")
        base = wid * RPW

        pltpu.sync_copy(idx_hbm.at[:, pl.ds(base, RPW)], cols_v)

        def build_chunk(c, carry):
            i = base + c * L + lax.iota(jnp.int32, L)
            ioff = ((i >> 3) << 16) + ((i & 7) << 7)

            def build_j(j, carry):
                col = cols_v[j, pl.ds(c * L, L)]
                flat_v[pl.ds(j * RPW + c * L, L)] = (
                    ioff + ((col >> 7) << 10) + (col & 127))
                return carry

            return lax.fori_loop(0, K1, build_j, carry)

        lax.fori_loop(0, RPW // L, build_chunk, 0)

        descs = [
            pltpu.async_copy(
                data_hbm.at[flat_v.at[pl.ds(m * CH, CH)]],
                gath_v.at[pl.ds(m * CH, CH)], sem)
            for m in range(NCHUNK)
        ]
        for d in descs:
            d.wait()

        def reduce_chunk(c, carry):
            def add_j(j, acc):
                return acc + gath_v[pl.ds(j * RPW + c * L, L)]

            acc = lax.fori_loop(1, K1, add_j, gath_v[pl.ds(c * L, L)])
            sums_v[pl.ds(c * L, L)] = acc
            return carry

        lax.fori_loop(0, RPW // L, reduce_chunk, 0)

        pltpu.sync_copy(sums_v, out_hbm.at[pl.ds(base, RPW)])

    return rowsum_kernel(data_phys, idx_t)


def _tc_softmax(a_raw):
    def body(x_ref, alpha_ref):
        x = x_ref[...]
        m = jnp.max(x)
        e = jnp.exp(x - m)
        alpha_ref[...] = e / jnp.sum(e)

    alpha = pl.pallas_call(
        body,
        out_shape=jax.ShapeDtypeStruct((8, N // 8), jnp.float32),
    )(a_raw.reshape(8, N // 8))
    return alpha.reshape(N)


def kernel(data_input, neighbor_indices):
    idx = neighbor_indices[:, :K1].astype(jnp.int32)
    idx_t = idx.T.reshape(K1, N)
    data_phys = (data_input
                 .reshape(N // 8, 8, N // 128, 128)
                 .transpose(0, 2, 1, 3)
                 .reshape(N * N))
    a_raw = _sc_row_sums(data_phys, idx_t)
    alpha = _tc_softmax(a_raw)
    return (alpha, a_raw)
